# Initial kernel scaffold; baseline (speedup 1.0000x reference)
import functools

import jax
import jax.numpy as jnp
from jax import lax
from jax.experimental import pallas as pl
from jax.experimental.pallas import tpu as pltpu
from jax.experimental.pallas import tpu_sc as plsc

N_NODES = 10000
N_EDGES = 320000
NUM_CORES = 2
NUM_SUBCORES = 16
NW = NUM_CORES * NUM_SUBCORES
PERW = N_EDGES // NW
CHUNK = 128
NFULL = PERW // CHUNK
TAIL = PERW - NFULL * CHUNK
RPW = N_NODES // NUM_SUBCORES

_MESH = plsc.VectorSubcoreMesh(core_axis_name="c", subcore_axis_name="s")


def _sc_degree(dst, zeros16):
    @functools.partial(
        pl.kernel,
        out_type=jax.ShapeDtypeStruct((NUM_CORES, N_NODES, 16), jnp.float32),
        mesh=_MESH,
        scratch_types=[
            pltpu.VMEM((CHUNK,), jnp.int32),
            pltpu.VMEM((TAIL,), jnp.int32),
            pltpu.VMEM((CHUNK, 16), jnp.float32),
            pltpu.VMEM((TAIL, 16), jnp.float32),
        ],
    )
    def k(dst_hbm, zeros_hbm, out_hbm, dst_v, tdst_v, ones_v, tones_v):
        cid = lax.axis_index("c")
        sid = lax.axis_index("s")
        acc = out_hbm.at[cid]
        for r in range(CHUNK):
            ones_v[r, :] = jnp.ones((16,), jnp.float32)
        for r in range(TAIL):
            tones_v[r, :] = jnp.ones((16,), jnp.float32)
        pltpu.sync_copy(zeros_hbm.at[pl.ds(sid * RPW, RPW)],
                        acc.at[pl.ds(sid * RPW, RPW)])
        plsc.subcore_barrier()
        ebase = cid * (N_EDGES // 2) + sid * PERW

        @pl.loop(0, NFULL)
        def _(i):
            pltpu.sync_copy(dst_hbm.at[pl.ds(ebase + i * CHUNK, CHUNK)], dst_v)
            pltpu.sync_copy(ones_v, acc.at[dst_v], add=True)

        t0 = ebase + NFULL * CHUNK
        pltpu.sync_copy(dst_hbm.at[pl.ds(t0, TAIL)], tdst_v)
        pltpu.sync_copy(tones_v, acc.at[tdst_v], add=True)

    return k(dst, zeros16)


def _make_sc_aggregate(D):
    @functools.partial(
        pl.kernel,
        out_type=jax.ShapeDtypeStruct((NUM_CORES, N_NODES, D), jnp.float32),
        mesh=_MESH,
        scratch_types=[
            pltpu.VMEM((CHUNK,), jnp.int32),
            pltpu.VMEM((CHUNK,), jnp.int32),
            pltpu.VMEM((CHUNK, D), jnp.float32),
            pltpu.VMEM((TAIL,), jnp.int32),
            pltpu.VMEM((TAIL,), jnp.int32),
            pltpu.VMEM((TAIL, D), jnp.float32),
            pltpu.VMEM_SHARED((N_NODES, D), jnp.float32),
            pltpu.SemaphoreType.DMA,
        ],
    )
    def k(y_hbm, src_hbm, dst_hbm, zeros_hbm, out_hbm,
          src_v, dst_v, rows_v, tsrc_v, tdst_v, trows_v, acc_sh, sem):
        cid = lax.axis_index("c")
        sid = lax.axis_index("s")
        pltpu.sync_copy(zeros_hbm.at[pl.ds(sid * RPW, RPW)],
                        acc_sh.at[pl.ds(sid * RPW, RPW)])
        plsc.subcore_barrier()
        ebase = cid * (N_EDGES // 2) + sid * PERW

        @pl.loop(0, NFULL)
        def _(i):
            e0 = ebase + i * CHUNK
            pltpu.sync_copy(src_hbm.at[pl.ds(e0, CHUNK)], src_v)
            pltpu.sync_copy(dst_hbm.at[pl.ds(e0, CHUNK)], dst_v)
            pltpu.async_copy(y_hbm.at[src_v], rows_v, sem).wait()
            pltpu.sync_copy(rows_v, acc_sh.at[dst_v], add=True)

        t0 = ebase + NFULL * CHUNK
        pltpu.sync_copy(src_hbm.at[pl.ds(t0, TAIL)], tsrc_v)
        pltpu.sync_copy(dst_hbm.at[pl.ds(t0, TAIL)], tdst_v)
        pltpu.async_copy(y_hbm.at[tsrc_v], trows_v, sem).wait()
        pltpu.sync_copy(trows_v, acc_sh.at[tdst_v], add=True)
        plsc.subcore_barrier()
        pltpu.sync_copy(acc_sh.at[pl.ds(sid * RPW, RPW)],
                        out_hbm.at[cid].at[pl.ds(sid * RPW, RPW)])

    return k


_sc_aggregate_128 = _make_sc_aggregate(128)
_sc_aggregate_64 = _make_sc_aggregate(64)

_BN = 1000


def _tc_matmul(x, W):
    n, kdim = x.shape
    h = W.shape[1]

    def body(x_ref, w_ref, o_ref):
        o_ref[...] = lax.dot_general(
            x_ref[...], w_ref[...], (((1,), (0,)), ((), ())),
            preferred_element_type=jnp.float32,
            precision=lax.Precision.HIGHEST)

    return pl.pallas_call(
        body,
        grid=(n // _BN,),
        in_specs=[pl.BlockSpec((_BN, kdim), lambda i: (i, 0)),
                  pl.BlockSpec((kdim, h), lambda i: (0, 0))],
        out_specs=pl.BlockSpec((_BN, h), lambda i: (i, 0)),
        out_shape=jax.ShapeDtypeStruct((n, h), jnp.float32),
    )(x, W)


def _tc_dis_scale(deg0, deg1, xw):
    n, h = xw.shape

    def body(d0_ref, d1_ref, xw_ref, y_ref, dis_ref):
        deg = d0_ref[...][:, :1] + d1_ref[...][:, :1] + 1.0
        dis = lax.rsqrt(deg)
        dis_ref[...] = dis
        y_ref[...] = xw_ref[...] * dis

    return pl.pallas_call(
        body,
        grid=(n // _BN,),
        in_specs=[pl.BlockSpec((_BN, 16), lambda i: (i, 0)),
                  pl.BlockSpec((_BN, 16), lambda i: (i, 0)),
                  pl.BlockSpec((_BN, h), lambda i: (i, 0))],
        out_specs=[pl.BlockSpec((_BN, h), lambda i: (i, 0)),
                   pl.BlockSpec((_BN, 1), lambda i: (i, 0))],
        out_shape=[jax.ShapeDtypeStruct((n, h), jnp.float32),
                   jax.ShapeDtypeStruct((n, 1), jnp.float32)],
    )(deg0, deg1, xw)


def _tc_layer2_fuse(a0, a1, y1, dis, b1, W2):
    n, h1 = y1.shape
    h2 = W2.shape[1]

    def body(a0_ref, a1_ref, y1_ref, dis_ref, b1_ref, w2_ref, y2_ref):
        dis = dis_ref[...]
        hidden = dis * (a0_ref[...] + a1_ref[...] + y1_ref[...]) + b1_ref[...]
        hidden = jnp.maximum(hidden, 0.0)
        y2_ref[...] = dis * lax.dot_general(
            hidden, w2_ref[...], (((1,), (0,)), ((), ())),
            preferred_element_type=jnp.float32,
            precision=lax.Precision.HIGHEST)

    return pl.pallas_call(
        body,
        grid=(n // _BN,),
        in_specs=[pl.BlockSpec((_BN, h1), lambda i: (i, 0)),
                  pl.BlockSpec((_BN, h1), lambda i: (i, 0)),
                  pl.BlockSpec((_BN, h1), lambda i: (i, 0)),
                  pl.BlockSpec((_BN, 1), lambda i: (i, 0)),
                  pl.BlockSpec((1, h1), lambda i: (0, 0)),
                  pl.BlockSpec((h1, h2), lambda i: (0, 0))],
        out_specs=pl.BlockSpec((_BN, h2), lambda i: (i, 0)),
        out_shape=jax.ShapeDtypeStruct((n, h2), jnp.float32),
    )(a0, a1, y1, dis, b1.reshape(1, h1), W2)


def _tc_layer2_post(a0, a1, y2, dis, b2):
    n, h = y2.shape

    def body(a0_ref, a1_ref, y2_ref, dis_ref, b2_ref, h_ref, ss_ref):
        i = pl.program_id(0)
        out = dis_ref[...] * (a0_ref[...] + a1_ref[...] + y2_ref[...]) \
            + b2_ref[...]
        h_ref[...] = out

        @pl.when(i == 0)
        def _():
            ss_ref[...] = jnp.zeros_like(ss_ref)

        ss_ref[...] += jnp.sum(out * out, axis=0, keepdims=True)

    return pl.pallas_call(
        body,
        grid=(n // _BN,),
        in_specs=[pl.BlockSpec((_BN, h), lambda i: (i, 0)),
                  pl.BlockSpec((_BN, h), lambda i: (i, 0)),
                  pl.BlockSpec((_BN, h), lambda i: (i, 0)),
                  pl.BlockSpec((_BN, 1), lambda i: (i, 0)),
                  pl.BlockSpec((1, h), lambda i: (0, 0))],
        out_specs=[pl.BlockSpec((_BN, h), lambda i: (i, 0)),
                   pl.BlockSpec((1, h), lambda i: (0, 0))],
        out_shape=[jax.ShapeDtypeStruct((n, h), jnp.float32),
                   jax.ShapeDtypeStruct((1, h), jnp.float32)],
    )(a0, a1, y2, dis, b2.reshape(1, h))


def _tc_colnorm_div(h2, ss):
    n, h = h2.shape

    def body(h_ref, ss_ref, o_ref):
        scale = 1.0 / jnp.maximum(jnp.sqrt(ss_ref[...]), 1e-12)
        o_ref[...] = h_ref[...] * scale

    return pl.pallas_call(
        body,
        grid=(n // _BN,),
        in_specs=[pl.BlockSpec((_BN, h), lambda i: (i, 0)),
                  pl.BlockSpec((1, h), lambda i: (0, 0))],
        out_specs=pl.BlockSpec((_BN, h), lambda i: (i, 0)),
        out_shape=jax.ShapeDtypeStruct((n, h), jnp.float32),
    )(h2, ss)


def kernel(x, edge_index, W1, b1, W2, b2):
    src = edge_index[0]
    dst = edge_index[1]
    zeros128 = jnp.zeros((N_NODES, 128), jnp.float32)
    zeros64 = jnp.zeros((N_NODES, 64), jnp.float32)
    zeros16 = jnp.zeros((N_NODES, 16), jnp.float32)

    degp = _sc_degree(dst, zeros16)
    xw1 = _tc_matmul(x, W1)
    y1, dis = _tc_dis_scale(degp[0], degp[1], xw1)

    agg1 = _sc_aggregate_128(y1, src, dst, zeros128)
    y2 = _tc_layer2_fuse(agg1[0], agg1[1], y1, dis, b1, W2)

    agg2 = _sc_aggregate_64(y2, src, dst, zeros64)
    h2, ss = _tc_layer2_post(agg2[0], agg2[1], y2, dis, b2)
    return _tc_colnorm_div(h2, ss)


# speedup vs baseline: 16.3586x; 16.3586x over previous
import dataclasses
import functools

import jax
import jax.numpy as jnp
from jax import lax
from jax.experimental import pallas as pl
from jax.experimental.pallas import tpu as pltpu
from jax.experimental.pallas import tpu_sc as plsc

N_NODES = 10000
N_EDGES = 320000
NUM_CORES = 2
NUM_SUBCORES = 16
NW = NUM_CORES * NUM_SUBCORES
PERW = N_EDGES // NW
CHUNK = 128
NFULL = PERW // CHUNK
TAIL = PERW - NFULL * CHUNK
RPW = 632
N_PAD = RPW * NUM_SUBCORES

_MESH = plsc.VectorSubcoreMesh(core_axis_name="c", subcore_axis_name="s")
_SC_PARAMS = pltpu.CompilerParams()
if "needs_layout_passes" in pltpu.CompilerParams.__dataclass_fields__:
    _SC_PARAMS = dataclasses.replace(_SC_PARAMS, needs_layout_passes=False)


def _sc_degree(dst):
    @functools.partial(
        pl.kernel,
        out_type=jax.ShapeDtypeStruct((NW, N_PAD), jnp.float32),
        mesh=_MESH,
        compiler_params=_SC_PARAMS,
        scratch_types=[
            pltpu.VMEM((PERW,), jnp.int32),
            pltpu.VMEM((N_PAD,), jnp.float32),
        ],
    )
    def k(dst_hbm, out_hbm, dst_v, hist_v):
        cid = lax.axis_index("c")
        sid = lax.axis_index("s")
        wid = cid * NUM_SUBCORES + sid

        @pl.loop(0, N_PAD // 16)
        def _(i):
            hist_v[pl.ds(i * 16, 16)] = jnp.zeros((16,), jnp.float32)

        ebase = cid * (N_EDGES // 2) + sid * PERW
        pltpu.sync_copy(dst_hbm.at[pl.ds(ebase, PERW)], dst_v)
        ones = jnp.ones((16,), jnp.float32)

        @pl.loop(0, PERW // 16)
        def _(i):
            idx = dst_v[pl.ds(i * 16, 16)]
            plsc.addupdate_scatter(hist_v, [idx], ones)

        pltpu.sync_copy(hist_v, out_hbm.at[wid])

    return k(dst)


def _tc_degsum(degp):
    nw, npad = degp.shape

    def body(p_ref, dis_ref):
        deg = jnp.sum(p_ref[...], axis=0, keepdims=True) + 1.0
        dis_ref[...] = lax.rsqrt(deg)

    return pl.pallas_call(
        body,
        grid=(1,),
        in_specs=[pl.BlockSpec((nw, npad), lambda i: (0, 0))],
        out_specs=pl.BlockSpec((1, npad), lambda i: (0, 0)),
        out_shape=jax.ShapeDtypeStruct((1, npad), jnp.float32),
    )(degp)


def _make_sc_aggregate(D):
    @functools.partial(
        pl.kernel,
        out_type=jax.ShapeDtypeStruct((NUM_CORES, N_PAD, D), jnp.float32),
        mesh=_MESH,
        scratch_types=[
            pltpu.VMEM((CHUNK,), jnp.int32),
            pltpu.VMEM((CHUNK,), jnp.int32),
            pltpu.VMEM((CHUNK, D), jnp.float32),
            pltpu.VMEM((TAIL,), jnp.int32),
            pltpu.VMEM((TAIL,), jnp.int32),
            pltpu.VMEM((TAIL, D), jnp.float32),
            pltpu.VMEM_SHARED((N_PAD, D), jnp.float32),
            pltpu.SemaphoreType.DMA,
        ],
    )
    def k(y_hbm, src_hbm, dst_hbm, zeros_hbm, out_hbm,
          src_v, dst_v, rows_v, tsrc_v, tdst_v, trows_v, acc_sh, sem):
        cid = lax.axis_index("c")
        sid = lax.axis_index("s")
        pltpu.sync_copy(zeros_hbm.at[pl.ds(sid * RPW, RPW)],
                        acc_sh.at[pl.ds(sid * RPW, RPW)])
        plsc.subcore_barrier()
        ebase = cid * (N_EDGES // 2) + sid * PERW

        @pl.loop(0, NFULL)
        def _(i):
            e0 = ebase + i * CHUNK
            pltpu.sync_copy(src_hbm.at[pl.ds(e0, CHUNK)], src_v)
            pltpu.sync_copy(dst_hbm.at[pl.ds(e0, CHUNK)], dst_v)
            pltpu.async_copy(y_hbm.at[src_v], rows_v, sem).wait()
            pltpu.sync_copy(rows_v, acc_sh.at[dst_v], add=True)

        t0 = ebase + NFULL * CHUNK
        pltpu.sync_copy(src_hbm.at[pl.ds(t0, TAIL)], tsrc_v)
        pltpu.sync_copy(dst_hbm.at[pl.ds(t0, TAIL)], tdst_v)
        pltpu.async_copy(y_hbm.at[tsrc_v], trows_v, sem).wait()
        pltpu.sync_copy(trows_v, acc_sh.at[tdst_v], add=True)
        plsc.subcore_barrier()
        pltpu.sync_copy(acc_sh.at[pl.ds(sid * RPW, RPW)],
                        out_hbm.at[cid].at[pl.ds(sid * RPW, RPW)])

    return k


_sc_aggregate_128 = _make_sc_aggregate(128)

_BN = 1000


def _tc_matmul(x, W):
    n, kdim = x.shape
    h = W.shape[1]

    def body(x_ref, w_ref, o_ref):
        o_ref[...] = lax.dot_general(
            x_ref[...], w_ref[...], (((1,), (0,)), ((), ())),
            preferred_element_type=jnp.float32,
            precision=lax.Precision.HIGHEST)

    return pl.pallas_call(
        body,
        grid=(n // _BN,),
        in_specs=[pl.BlockSpec((_BN, kdim), lambda i: (i, 0)),
                  pl.BlockSpec((kdim, h), lambda i: (0, 0))],
        out_specs=pl.BlockSpec((_BN, h), lambda i: (i, 0)),
        out_shape=jax.ShapeDtypeStruct((n, h), jnp.float32),
    )(x, W)


def _tc_dis_scale(dis, xw):
    n, h = xw.shape

    def body(dis_ref, xw_ref, y_ref):
        y_ref[...] = xw_ref[...] * dis_ref[...]

    return pl.pallas_call(
        body,
        grid=(n // _BN,),
        in_specs=[pl.BlockSpec((_BN, 1), lambda i: (i, 0)),
                  pl.BlockSpec((_BN, h), lambda i: (i, 0))],
        out_specs=pl.BlockSpec((_BN, h), lambda i: (i, 0)),
        out_shape=jax.ShapeDtypeStruct((n, h), jnp.float32),
    )(dis, xw)


def _tc_layer2_fuse(a0, a1, y1, dis, b1, W2):
    n, h1 = y1.shape
    h2 = W2.shape[1]

    def body(a0_ref, a1_ref, y1_ref, dis_ref, b1_ref, w2_ref, y2_ref):
        dis = dis_ref[...]
        hidden = dis * (a0_ref[...] + a1_ref[...] + y1_ref[...]) + b1_ref[...]
        hidden = jnp.maximum(hidden, 0.0)
        prod = dis * lax.dot_general(
            hidden, w2_ref[...], (((1,), (0,)), ((), ())),
            preferred_element_type=jnp.float32,
            precision=lax.Precision.HIGHEST)
        y2_ref[...] = jnp.concatenate(
            [prod, jnp.zeros_like(prod)], axis=1)

    return pl.pallas_call(
        body,
        grid=(n // _BN,),
        in_specs=[pl.BlockSpec((_BN, h1), lambda i: (i, 0)),
                  pl.BlockSpec((_BN, h1), lambda i: (i, 0)),
                  pl.BlockSpec((_BN, h1), lambda i: (i, 0)),
                  pl.BlockSpec((_BN, 1), lambda i: (i, 0)),
                  pl.BlockSpec((1, h1), lambda i: (0, 0)),
                  pl.BlockSpec((h1, h2), lambda i: (0, 0))],
        out_specs=pl.BlockSpec((_BN, 2 * h2), lambda i: (i, 0)),
        out_shape=jax.ShapeDtypeStruct((n, 2 * h2), jnp.float32),
    )(a0, a1, y1, dis, b1.reshape(1, h1), W2)


def _tc_layer2_post(a0, a1, y2, dis, b2):
    n, w = y2.shape
    h = w // 2

    def body(a0_ref, a1_ref, y2_ref, dis_ref, b2_ref, h_ref, ss_ref):
        i = pl.program_id(0)
        s = (a0_ref[...] + a1_ref[...] + y2_ref[...])[:, :h]
        out = dis_ref[...] * s + b2_ref[...]
        h_ref[...] = out

        @pl.when(i == 0)
        def _():
            ss_ref[...] = jnp.zeros_like(ss_ref)

        ss_ref[...] += jnp.sum(out * out, axis=0, keepdims=True)

    return pl.pallas_call(
        body,
        grid=(n // _BN,),
        in_specs=[pl.BlockSpec((_BN, w), lambda i: (i, 0)),
                  pl.BlockSpec((_BN, w), lambda i: (i, 0)),
                  pl.BlockSpec((_BN, w), lambda i: (i, 0)),
                  pl.BlockSpec((_BN, 1), lambda i: (i, 0)),
                  pl.BlockSpec((1, h), lambda i: (0, 0))],
        out_specs=[pl.BlockSpec((_BN, h), lambda i: (i, 0)),
                   pl.BlockSpec((1, h), lambda i: (0, 0))],
        out_shape=[jax.ShapeDtypeStruct((n, h), jnp.float32),
                   jax.ShapeDtypeStruct((1, h), jnp.float32)],
    )(a0, a1, y2, dis, b2.reshape(1, h))


def _tc_colnorm_div(h2, ss):
    n, h = h2.shape

    def body(h_ref, ss_ref, o_ref):
        scale = 1.0 / jnp.maximum(jnp.sqrt(ss_ref[...]), 1e-12)
        o_ref[...] = h_ref[...] * scale

    return pl.pallas_call(
        body,
        grid=(n // _BN,),
        in_specs=[pl.BlockSpec((_BN, h), lambda i: (i, 0)),
                  pl.BlockSpec((1, h), lambda i: (0, 0))],
        out_specs=pl.BlockSpec((_BN, h), lambda i: (i, 0)),
        out_shape=jax.ShapeDtypeStruct((n, h), jnp.float32),
    )(h2, ss)


def kernel(x, edge_index, W1, b1, W2, b2):
    src = edge_index[0]
    dst = edge_index[1]
    zeros128 = jnp.zeros((N_PAD, 128), jnp.float32)

    degp = _sc_degree(dst)
    xw1 = _tc_matmul(x, W1)
    dis_row = _tc_degsum(degp)
    dis = dis_row[0, :N_NODES].reshape(N_NODES, 1)
    y1 = _tc_dis_scale(dis, xw1)

    agg1 = _sc_aggregate_128(y1, src, dst, zeros128)
    y2 = _tc_layer2_fuse(agg1[0, :N_NODES], agg1[1, :N_NODES], y1, dis, b1, W2)

    agg2 = _sc_aggregate_128(y2, src, dst, zeros128)
    h2, ss = _tc_layer2_post(agg2[0, :N_NODES], agg2[1, :N_NODES], y2, dis, b2)
    return _tc_colnorm_div(h2, ss)
